# baseline (speedup 1.0000x reference)
import jax
import jax.numpy as jnp
from jax.experimental import pallas as pl

_N = 4096
_D = 64
_BM = 256


def _mm_body(w_ref, x_ref, o_ref):
    o_ref[...] = jnp.dot(w_ref[...], x_ref[...],
                         preferred_element_type=jnp.float32)


def _matmul_af(W, act):
    return pl.pallas_call(
        _mm_body,
        grid=(_N // _BM,),
        in_specs=[
            pl.BlockSpec((_BM, _N), lambda i: (i, 0)),
            pl.BlockSpec((_N, _D), lambda i: (0, 0)),
        ],
        out_specs=pl.BlockSpec((_BM, _D), lambda i: (i, 0)),
        out_shape=jax.ShapeDtypeStruct((_N, _D), jnp.float32),
    )(W, act)


def kernel(currents, targets, activities_features, cases_features, W):
    af = _matmul_af(W, activities_features)
    hc = jnp.take(activities_features, currents, axis=0)
    ht = jnp.take(activities_features, targets, axis=0)
    diff = hc + cases_features - ht
    dist = jnp.sqrt(jnp.sum(diff * diff, axis=1))
    vals = jnp.exp(-dist)
    den = jnp.zeros((_N,), jnp.float32).at[targets].add(vals)
    contrib = vals[:, None] * jnp.take(af, currents, axis=0)
    num = jnp.zeros((_N, _D), jnp.float32).at[targets].add(contrib)
    return num / (den[:, None] + 1e-12) + af


# speedup vs baseline: 3.5649x; 3.5649x over previous
import functools

import jax
import jax.numpy as jnp
from jax import lax
from jax.experimental import pallas as pl
from jax.experimental.pallas import tpu as pltpu
from jax.experimental.pallas import tpu_sc as plsc

_N = 4096
_D = 64
_E = 262144
_NC = 2
_NS = 16
_NW = _NC * _NS
_EPW = _E // _NW
_CH1 = 512
_CH2 = 1024
_BM = 256


def _mm_body(w_ref, x_ref, o_ref):
    o_ref[...] = jnp.dot(w_ref[...], x_ref[...],
                         preferred_element_type=jnp.float32)


def _matmul_af(W, act):
    return pl.pallas_call(
        _mm_body,
        grid=(_N // _BM,),
        in_specs=[
            pl.BlockSpec((_BM, _N), lambda i: (i, 0)),
            pl.BlockSpec((_N, _D), lambda i: (0, 0)),
        ],
        out_specs=pl.BlockSpec((_BM, _D), lambda i: (i, 0)),
        out_shape=jax.ShapeDtypeStruct((_N, _D), jnp.float32),
    )(W, act)


def _combine_body(num_ref, den_ref, af_ref, o_ref):
    den = den_ref[0, :] + den_ref[1, :]
    num = num_ref[0] + num_ref[1]
    o_ref[...] = num / (den[:, None] + 1e-12) + af_ref[...]


def _combine(num2, den2, af):
    blk = 512
    return pl.pallas_call(
        _combine_body,
        grid=(_N // blk,),
        in_specs=[
            pl.BlockSpec((2, blk, _D), lambda i: (0, i, 0)),
            pl.BlockSpec((2, blk), lambda i: (0, i)),
            pl.BlockSpec((blk, _D), lambda i: (i, 0)),
        ],
        out_specs=pl.BlockSpec((blk, _D), lambda i: (i, 0)),
        out_shape=jax.ShapeDtypeStruct((_N, _D), jnp.float32),
    )(num2, den2, af)


def _rsqrt_newton(ss):
    i = plsc.bitcast(ss, jnp.int32)
    i = jnp.int32(0x5F3759DF) - (i >> 1)
    y = plsc.bitcast(i, jnp.float32)
    for _ in range(3):
        y = y * (1.5 - 0.5 * ss * y * y)
    return y


def _stage1_body(c2, t2, act, cases, v_out, cv, tv, hc, ht, cs, vv, sem_c,
                 sem_t):
    cid = lax.axis_index("c")
    sid = lax.axis_index("s")
    wid = sid * _NC + cid
    col16 = lax.iota(jnp.int32, 16)
    cols = [col16 + (16 * k) for k in range(4)]

    def chunk_body(sub, _):
        off = pl.multiple_of(wid * _EPW + sub * (2 * _CH1), 1024)
        row_off = pl.multiple_of(off // 128, 8)
        pltpu.sync_copy(c2.at[pl.ds(row_off, 2 * _CH1 // 128)], cv)
        pltpu.sync_copy(t2.at[pl.ds(row_off, 2 * _CH1 // 128)], tv)
        for h in range(2):
            hoff = pl.multiple_of(off + h * _CH1, 512)
            cps = []
            for j in range(_CH1 // 128):
                r = h * (_CH1 // 128) + j
                cps.append(pltpu.async_copy(
                    act.at[cv.at[r]], hc.at[pl.ds(j * 128, 128)], sem_c))
                cps.append(pltpu.async_copy(
                    act.at[tv.at[r]], ht.at[pl.ds(j * 128, 128)], sem_t))
            pltpu.sync_copy(cases.at[pl.ds(hoff, _CH1)], cs)
            for cp in cps:
                cp.wait()

            def group_body(g, _):
                rows = g * 16 + col16

                def d_body(d, acc):
                    dcol = jnp.broadcast_to(d, (16,))
                    a = plsc.load_gather(hc, [rows, dcol])
                    b = plsc.load_gather(ht, [rows, dcol])
                    c = plsc.load_gather(cs, [rows, dcol])
                    df = a + c - b
                    return acc + df * df

                acc = lax.fori_loop(0, _D, d_body,
                                    jnp.zeros((16,), jnp.float32), unroll=8)
                ss = jnp.maximum(acc, 1e-30)
                dist = acc * _rsqrt_newton(ss)
                vv[pl.ds(g * 16, 16)] = jnp.exp(-dist)
                return 0

            lax.fori_loop(0, _CH1 // 16, group_body, 0)
            pltpu.sync_copy(vv, v_out.at[pl.ds(hoff, _CH1)])
        return 0

    lax.fori_loop(0, _EPW // (2 * _CH1), chunk_body, 0)


def _stage2_body(c2, t2, v2, af, z2d, z1d, num_out, den_out, cv, tv, vv, rows,
                 num_sh, den_sh, sem):
    cid = lax.axis_index("c")
    sid = lax.axis_index("s")
    wid = sid * _NC + cid
    npc = _N // _NS
    srow = pl.multiple_of(sid * npc, 8)

    pltpu.sync_copy(z2d.at[pl.ds(srow, npc)],
                    num_sh.at[pl.ds(srow, npc)])
    pltpu.sync_copy(z1d.at[pl.ds(srow, npc)],
                    den_sh.at[pl.ds(srow, npc)])
    plsc.subcore_barrier()

    nrow = _CH2 // 128

    def chunk_body(sub, _):
        off = pl.multiple_of(wid * _EPW + sub * _CH2, 1024)
        row_off = pl.multiple_of(off // 128, 8)
        pltpu.sync_copy(c2.at[pl.ds(row_off, nrow)], cv)
        pltpu.sync_copy(t2.at[pl.ds(row_off, nrow)], tv)
        pltpu.sync_copy(v2.at[pl.ds(row_off, nrow)], vv)
        cps = []
        for j in range(nrow):
            cps.append(pltpu.async_copy(
                af.at[cv.at[j]], rows.at[pl.ds(j * 128, 128)], sem))
        for cp in cps:
            cp.wait()

        def scale_body(g, _):
            v16 = vv[g // 8, pl.ds((g % 8) * 16, 16)]
            for j in range(16):
                e = g * 16 + j
                vb = jnp.broadcast_to(v16[j], (16,))
                for k in range(4):
                    sl = pl.ds(k * 16, 16)
                    rows[e, sl] = rows[e, sl] * vb
            return 0

        lax.fori_loop(0, _CH2 // 16, scale_body, 0)
        for j in range(nrow):
            pltpu.sync_copy(rows.at[pl.ds(j * 128, 128)],
                            num_sh.at[tv.at[j]], add=True)
            pltpu.sync_copy(vv.at[j], den_sh.at[tv.at[j]], add=True)
        return 0

    lax.fori_loop(0, _EPW // _CH2, chunk_body, 0)
    plsc.subcore_barrier()
    pltpu.sync_copy(num_sh.at[pl.ds(srow, npc)],
                    num_out.at[cid, pl.ds(srow, npc)])
    pltpu.sync_copy(den_sh.at[pl.ds(srow, npc)],
                    den_out.at[cid, pl.ds(srow, npc)])


def _edge_vals(currents2, targets2, act, cases):
    mesh = plsc.VectorSubcoreMesh(core_axis_name="c", subcore_axis_name="s")
    f = pl.kernel(
        _stage1_body,
        out_type=jax.ShapeDtypeStruct((_E,), jnp.float32),
        mesh=mesh,
        compiler_params=pltpu.CompilerParams(needs_layout_passes=False, use_tc_tiling_on_sc=False),
        scratch_types=[
            pltpu.VMEM((2 * _CH1 // 128, 128), jnp.int32),
            pltpu.VMEM((2 * _CH1 // 128, 128), jnp.int32),
            pltpu.VMEM((_CH1, _D), jnp.float32),
            pltpu.VMEM((_CH1, _D), jnp.float32),
            pltpu.VMEM((_CH1, _D), jnp.float32),
            pltpu.VMEM((_CH1,), jnp.float32),
            pltpu.SemaphoreType.DMA,
            pltpu.SemaphoreType.DMA,
        ],
    )
    return f(currents2, targets2, act, cases)


def _segment_sums(currents2, targets2, v2, af):
    mesh = plsc.VectorSubcoreMesh(core_axis_name="c", subcore_axis_name="s")
    z2d = jnp.zeros((_N, _D), jnp.float32)
    z1d = jnp.zeros((_N,), jnp.float32)
    f = pl.kernel(
        _stage2_body,
        out_type=(jax.ShapeDtypeStruct((_NC, _N, _D), jnp.float32),
                  jax.ShapeDtypeStruct((_NC, _N), jnp.float32)),
        mesh=mesh,
        compiler_params=pltpu.CompilerParams(needs_layout_passes=False, use_tc_tiling_on_sc=False),
        scratch_types=[
            pltpu.VMEM((_CH2 // 128, 128), jnp.int32),
            pltpu.VMEM((_CH2 // 128, 128), jnp.int32),
            pltpu.VMEM((_CH2 // 128, 128), jnp.float32),
            pltpu.VMEM((_CH2, _D), jnp.float32),
            pltpu.VMEM_SHARED((_N, _D), jnp.float32),
            pltpu.VMEM_SHARED((_N,), jnp.float32),
            pltpu.SemaphoreType.DMA,
        ],
    )
    return f(currents2, targets2, v2, af, z2d, z1d)


def kernel(currents, targets, activities_features, cases_features, W):
    c2 = currents.reshape(_E // 128, 128)
    t2 = targets.reshape(_E // 128, 128)
    af = _matmul_af(W, activities_features)
    v = _edge_vals(c2, t2, activities_features, cases_features)
    num2, den2 = _segment_sums(c2, t2, v.reshape(_E // 128, 128), af)
    return _combine(num2, den2, af)
